# Initial kernel scaffold; baseline (speedup 1.0000x reference)
import functools

import jax
import jax.numpy as jnp
from jax import lax
from jax.experimental import pallas as pl
from jax.experimental.pallas import tpu as pltpu
from jax.experimental.pallas import tpu_sc as plsc

B = 4096
V = 100000
D = 32
T = 26
NUM = 13

NC, NS = 2, 16
NW = NC * NS
ROWS = B * T
RPW = ROWS // NW
CHUNK = 128
NCH = RPW // CHUNK

_sc_mesh = plsc.VectorSubcoreMesh(core_axis_name="c", subcore_axis_name="s")


@functools.partial(
    pl.kernel,
    out_type=jax.ShapeDtypeStruct((ROWS, D), jnp.float32),
    mesh=_sc_mesh,
    scratch_types=[
        pltpu.VMEM((NCH, CHUNK), jnp.int32),
        pltpu.VMEM((RPW, D), jnp.float32),
        pltpu.SemaphoreType.DMA,
    ],
)
def _sc_gather(table_hbm, gidx_hbm, out_hbm, idx_v, rows_v, sem):
    wid = lax.axis_index("s") * NC + lax.axis_index("c")
    pltpu.sync_copy(gidx_hbm.at[pl.ds(wid * NCH, NCH)], idx_v)
    copies = [
        pltpu.async_copy(
            table_hbm.at[idx_v.at[j]],
            rows_v.at[pl.ds(j * CHUNK, CHUNK)],
            sem,
        )
        for j in range(NCH)
    ]
    for c in copies:
        c.wait()
    pltpu.sync_copy(rows_v, out_hbm.at[pl.ds(wid * RPW, RPW)])


def _mlp_body(emb_ref, num_ref, w1e_ref, w1n_ref, b1_ref, w2_ref, b2_ref,
              w3_ref, b3_ref, w4_ref, b4_ref, out_ref):
    x1 = jnp.dot(emb_ref[...], w1e_ref[...], preferred_element_type=jnp.float32)
    x1 = x1 + jnp.dot(num_ref[...], w1n_ref[...],
                      preferred_element_type=jnp.float32)
    h = jnp.maximum(x1 + b1_ref[...], 0.0)
    h = jnp.maximum(
        jnp.dot(h, w2_ref[...], preferred_element_type=jnp.float32)
        + b2_ref[...], 0.0)
    h = jnp.maximum(
        jnp.dot(h, w3_ref[...], preferred_element_type=jnp.float32)
        + b3_ref[...], 0.0)
    out_ref[...] = (
        jnp.dot(h, w4_ref[...], preferred_element_type=jnp.float32)
        + b4_ref[...])


def _mlp(emb, num, w1e, w1n, b1, w2, b2, w3, b3, w4, b4, *, interpret=False):
    bb = 512
    grid = B // bb
    full = lambda shape: pl.BlockSpec(shape, lambda i: (0, 0))
    return pl.pallas_call(
        _mlp_body,
        grid=(grid,),
        in_specs=[
            pl.BlockSpec((bb, T * D), lambda i: (i, 0)),
            pl.BlockSpec((bb, NUM), lambda i: (i, 0)),
            full((T * D, 512)),
            full((NUM, 512)),
            full((1, 512)),
            full((512, 256)),
            full((1, 256)),
            full((256, 128)),
            full((1, 128)),
            full((128, 1)),
            full((1, 1)),
        ],
        out_specs=pl.BlockSpec((bb, 1), lambda i: (i, 0)),
        out_shape=jax.ShapeDtypeStruct((B, 1), jnp.float32),
        interpret=interpret,
    )(emb, num, w1e, w1n, b1, w2, b2, w3, b3, w4, b4)


def kernel(numerical_features, cat_features, tables, W1, b1, W2, b2, W3, b3,
           W4, b4):
    cat = cat_features.reshape(T, B).astype(jnp.int32)
    gidx = cat.T + (jnp.arange(T, dtype=jnp.int32) * V)[None, :]
    gidx = gidx.reshape(NW * NCH, CHUNK)
    flat_table = tables.reshape(T * V, D)

    emb = _sc_gather(flat_table, gidx)
    emb = emb.reshape(B, T * D)

    w1e = W1[: T * D]
    w1n = W1[T * D:]
    return _mlp(emb, numerical_features, w1e, w1n, b1.reshape(1, 512),
                W2, b2.reshape(1, 256), W3, b3.reshape(1, 128),
                W4, b4.reshape(1, 1))


# speedup vs baseline: 2.2065x; 2.2065x over previous
import functools

import jax
import jax.numpy as jnp
from jax import lax
from jax.experimental import pallas as pl
from jax.experimental.pallas import tpu as pltpu
from jax.experimental.pallas import tpu_sc as plsc

B = 4096
V = 100000
D = 32
T = 26
NUM = 13

NC, NS = 2, 16
NW = NC * NS
ROWS = B * T
RPW = ROWS // NW
CHUNK = 128
NCH = RPW // CHUNK

@functools.lru_cache(maxsize=None)
def _make_sc_gather():
    mesh = plsc.VectorSubcoreMesh(
        core_axis_name="c", subcore_axis_name="s",
        num_cores=NC, num_subcores=NS)

    @functools.partial(
        pl.kernel,
        out_type=jax.ShapeDtypeStruct((ROWS, D), jnp.float32),
        mesh=mesh,
        scratch_types=[
            pltpu.VMEM((NCH, CHUNK), jnp.int32),
            pltpu.VMEM((RPW, D), jnp.float32),
            pltpu.SemaphoreType.DMA,
        ],
        compiler_params=pltpu.CompilerParams(use_tc_tiling_on_sc=False),
    )
    def _sc_gather(table_hbm, gidx_hbm, out_hbm, idx_v, rows_v, sem):
        wid = lax.axis_index("s") * NC + lax.axis_index("c")
        pltpu.sync_copy(gidx_hbm.at[wid], idx_v)
        copies = [
            pltpu.async_copy(
                table_hbm.at[idx_v.at[j]],
                rows_v.at[pl.ds(j * CHUNK, CHUNK)],
                sem,
            )
            for j in range(NCH)
        ]
        for c in copies:
            c.wait()
        pltpu.sync_copy(rows_v, out_hbm.at[pl.ds(wid * RPW, RPW)])

    return _sc_gather


def _mlp_body(emb_ref, num_ref, w1e_ref, w1n_ref, b1_ref, w2_ref, b2_ref,
              w3_ref, b3_ref, w4_ref, b4_ref, out_ref):
    x1 = jnp.dot(emb_ref[...], w1e_ref[...], preferred_element_type=jnp.float32)
    x1 = x1 + jnp.dot(num_ref[...], w1n_ref[...],
                      preferred_element_type=jnp.float32)
    h = jnp.maximum(x1 + b1_ref[...], 0.0)
    h = jnp.maximum(
        jnp.dot(h, w2_ref[...], preferred_element_type=jnp.float32)
        + b2_ref[...], 0.0)
    h = jnp.maximum(
        jnp.dot(h, w3_ref[...], preferred_element_type=jnp.float32)
        + b3_ref[...], 0.0)
    out_ref[...] = (
        jnp.dot(h, w4_ref[...], preferred_element_type=jnp.float32)
        + b4_ref[...])


def _mlp(emb, num, w1e, w1n, b1, w2, b2, w3, b3, w4, b4, *, interpret=False):
    bb = 512
    grid = B // bb
    full = lambda shape: pl.BlockSpec(shape, lambda i: (0, 0))
    return pl.pallas_call(
        _mlp_body,
        grid=(grid,),
        in_specs=[
            pl.BlockSpec((bb, T * D), lambda i: (i, 0)),
            pl.BlockSpec((bb, NUM), lambda i: (i, 0)),
            full((T * D, 512)),
            full((NUM, 512)),
            full((1, 512)),
            full((512, 256)),
            full((1, 256)),
            full((256, 128)),
            full((1, 128)),
            full((128, 1)),
            full((1, 1)),
        ],
        out_specs=pl.BlockSpec((bb, 1), lambda i: (i, 0)),
        out_shape=jax.ShapeDtypeStruct((B, 1), jnp.float32),
        interpret=interpret,
    )(emb, num, w1e, w1n, b1, w2, b2, w3, b3, w4, b4)


def kernel(numerical_features, cat_features, tables, W1, b1, W2, b2, W3, b3,
           W4, b4):
    cat = cat_features.reshape(T, B).astype(jnp.int32)
    gidx = cat.T + (jnp.arange(T, dtype=jnp.int32) * V)[None, :]
    gidx = gidx.reshape(NW, NCH, CHUNK)
    flat_table = tables.reshape(T * V, D)

    emb = _make_sc_gather()(flat_table, gidx)
    emb = emb.reshape(B, T * D)

    w1e = W1[: T * D]
    w1n = W1[T * D:]
    return _mlp(emb, numerical_features, w1e, w1n, b1.reshape(1, 512),
                W2, b2.reshape(1, 256), W3, b3.reshape(1, 128),
                W4, b4.reshape(1, 1))
